# Initial kernel scaffold; baseline (speedup 1.0000x reference)
import jax
import jax.numpy as jnp
from jax.experimental import pallas as pl


def kernel(x, edge_index, W, b, att_l, att_r):
    raise NotImplementedError("write your pallas kernel here")


# speedup vs baseline: 4.9324x; 4.9324x over previous
import functools

import jax
import jax.numpy as jnp
from jax import lax
from jax.experimental import pallas as pl
from jax.experimental.pallas import tpu as pltpu
from jax.experimental.pallas import tpu_sc as plsc

N_NODES = 10000
N_EDGES = 320000
IN_CH = 128
OUT_CH = 32
HEADS = 4

NC = 2
NS = 16
NW = NC * NS
E_PER_W = N_EDGES // NW
SUB = 2000
N_SUB = E_PER_W // SUB
LANES = 16

ROW_BLOCK = 2000


def _alpha_body(x_ref, w_ref, b_ref, attl_ref, attr_ref, al_ref, ar_ref):
    x = x_ref[...]
    h = lax.dot_general(x, w_ref[...], (((1,), (1,)), ((), ())),
                        preferred_element_type=jnp.float32,
                        precision=lax.Precision.HIGHEST) + b_ref[...]
    row = lax.broadcasted_iota(jnp.int32, (IN_CH, HEADS), 0)
    col = lax.broadcasted_iota(jnp.int32, (IN_CH, HEADS), 1)
    seg = (row >= col * OUT_CH) & (row < (col + 1) * OUT_CH)
    sl = jnp.where(seg, jnp.broadcast_to(attl_ref[...], (IN_CH, HEADS)), 0.0)
    sr = jnp.where(seg, jnp.broadcast_to(attr_ref[...], (IN_CH, HEADS)), 0.0)
    al_ref[...] = lax.dot_general(h, sl, (((1,), (0,)), ((), ())),
                                  preferred_element_type=jnp.float32)
    ar_ref[...] = lax.dot_general(h, sr, (((1,), (0,)), ((), ())),
                                  preferred_element_type=jnp.float32)


_alpha_call = pl.pallas_call(
    _alpha_body,
    grid=(N_NODES // ROW_BLOCK,),
    in_specs=[
        pl.BlockSpec((ROW_BLOCK, IN_CH), lambda i: (i, 0)),
        pl.BlockSpec((IN_CH, IN_CH), lambda i: (0, 0)),
        pl.BlockSpec((1, IN_CH), lambda i: (0, 0)),
        pl.BlockSpec((IN_CH, 1), lambda i: (0, 0)),
        pl.BlockSpec((IN_CH, 1), lambda i: (0, 0)),
    ],
    out_specs=[
        pl.BlockSpec((ROW_BLOCK, HEADS), lambda i: (i, 0)),
        pl.BlockSpec((ROW_BLOCK, HEADS), lambda i: (i, 0)),
    ],
    out_shape=[
        jax.ShapeDtypeStruct((N_NODES, HEADS), jnp.float32),
        jax.ShapeDtypeStruct((N_NODES, HEADS), jnp.float32),
    ],
)


def _edge_body(al_hbm, ar_hbm, src_hbm, dst_hbm, out_hbm,
               al_v, ar_v, src_v, dst_v, out_v):
    wid = lax.axis_index("s") * NC + lax.axis_index("c")
    pltpu.sync_copy(al_hbm, al_v)
    pltpu.sync_copy(ar_hbm, ar_v)
    lane = lax.iota(jnp.int32, LANES)

    for s_idx in range(N_SUB):
        base = wid * E_PER_W + s_idx * SUB
        pltpu.sync_copy(src_hbm.at[pl.ds(base, SUB)], src_v)
        pltpu.sync_copy(dst_hbm.at[pl.ds(base, SUB)], dst_v)

        def body(j, carry):
            sv = src_v[pl.ds(j * LANES, LANES)] * HEADS
            dv = dst_v[pl.ds(j * LANES, LANES)] * HEADS
            obase = j * (LANES * HEADS)
            oidx = obase + lane * HEADS
            for hd in range(HEADS):
                a = plsc.load_gather(al_v, [sv + hd])
                r = plsc.load_gather(ar_v, [dv + hd])
                v = a + r
                res = jnp.where(v >= 0.0, v, v * jnp.float32(0.01))
                plsc.store_scatter(out_v, [oidx + hd], res)
            return carry

        lax.fori_loop(0, SUB // LANES, body, 0)
        pltpu.sync_copy(out_v, out_hbm.at[pl.ds(base * HEADS, SUB * HEADS)])


@functools.cache
def _edge_kernel():
    return pl.kernel(
        _edge_body,
        mesh=plsc.VectorSubcoreMesh(core_axis_name="c", subcore_axis_name="s",
                                    num_cores=NC, num_subcores=NS),
        compiler_params=pltpu.CompilerParams(needs_layout_passes=False),
        out_type=jax.ShapeDtypeStruct((N_EDGES * HEADS,), jnp.float32),
        scratch_types=[
            pltpu.VMEM((N_NODES * HEADS,), jnp.float32),
            pltpu.VMEM((N_NODES * HEADS,), jnp.float32),
            pltpu.VMEM((SUB,), jnp.int32),
            pltpu.VMEM((SUB,), jnp.int32),
            pltpu.VMEM((SUB * HEADS,), jnp.float32),
        ],
    )


def kernel(x, edge_index, W, b, att_l, att_r):
    src = edge_index[0].astype(jnp.int32)
    dst = edge_index[1].astype(jnp.int32)
    alpha_l, alpha_r = _alpha_call(
        x, W, b.reshape(1, IN_CH),
        att_l.reshape(IN_CH, 1), att_r.reshape(IN_CH, 1))
    out_flat = _edge_kernel()(alpha_l.reshape(-1), alpha_r.reshape(-1), src, dst)
    return out_flat.reshape(N_EDGES, HEADS)
